# baseline (speedup 1.0000x reference)
import jax
import jax.numpy as jnp
from jax.experimental import pallas as pl
from jax.experimental.pallas import tpu as pltpu

E = 8
D = 4096
T = 128
KB = 512
KT = D // KB
STEPS = E * KT
NBUF = 4


def _moe_body(x_ref, gw_ref, w_hbm, out_ref, comb_ref, buf_ref, sem):
    logits = jnp.dot(x_ref[...], gw_ref[...],
                     preferred_element_type=jnp.float32)
    eio = jax.lax.broadcasted_iota(jnp.int32, (T, E), 1)
    big = jnp.int32(E)
    m1 = jnp.max(logits, axis=1, keepdims=True)
    i1 = jnp.min(jnp.where(logits == m1, eio, big), axis=1, keepdims=True)
    sel1 = eio == i1
    masked = jnp.where(sel1, -jnp.inf, logits)
    m2 = jnp.max(masked, axis=1, keepdims=True)
    i2 = jnp.min(jnp.where(masked == m2, eio, big), axis=1, keepdims=True)
    sel2 = eio == i2
    t = jnp.exp(m2 - m1)
    w1 = 1.0 / (1.0 + t)
    comb_ref[...] = jnp.where(sel1, w1, 0.0) + jnp.where(sel2, 1.0 - w1, 0.0)
    out_ref[...] = jnp.zeros_like(out_ref)

    def start_copy(g):
        s = jax.lax.rem(g, NBUF)
        e = jax.lax.div(g, KT)
        k = jax.lax.rem(g, KT)
        pltpu.make_async_copy(
            w_hbm.at[e, pl.ds(k * KB, KB), :],
            buf_ref.at[s],
            sem.at[s],
        ).start()

    for g in range(NBUF - 1):
        start_copy(g)

    def step(g, _):
        s = jax.lax.rem(g, NBUF)
        e = jax.lax.div(g, KT)
        k = jax.lax.rem(g, KT)

        @pl.when(g + NBUF - 1 < STEPS)
        def _():
            start_copy(g + NBUF - 1)

        pltpu.make_async_copy(
            w_hbm.at[e, pl.ds(k * KB, KB), :],
            buf_ref.at[s],
            sem.at[s],
        ).wait()

        ce = jnp.sum(comb_ref[...] * (eio[:1] == e).astype(jnp.float32),
                     axis=1, keepdims=True)
        xe = x_ref[:, pl.ds(k * KB, KB)] * ce
        out_ref[...] += jnp.dot(xe, buf_ref[s],
                                precision=jax.lax.Precision.DEFAULT,
                                preferred_element_type=jnp.float32)
        return 0

    jax.lax.fori_loop(0, STEPS, step, 0)


def kernel(x, gate_w, expert_w):
    return pl.pallas_call(
        _moe_body,
        in_specs=[
            pl.BlockSpec((T, D), lambda: (0, 0)),
            pl.BlockSpec((D, E), lambda: (0, 0)),
            pl.BlockSpec(memory_space=pltpu.HBM),
        ],
        out_specs=pl.BlockSpec((T, D), lambda: (0, 0)),
        out_shape=jax.ShapeDtypeStruct((T, D), jnp.float32),
        scratch_shapes=[
            pltpu.VMEM((T, E), jnp.float32),
            pltpu.VMEM((NBUF, KB, D), jnp.float32),
            pltpu.SemaphoreType.DMA((NBUF,)),
        ],
    )(x, gate_w, expert_w)


# speedup vs baseline: 1.0269x; 1.0269x over previous
import jax
import jax.numpy as jnp
from jax.experimental import pallas as pl
from jax.experimental.pallas import tpu as pltpu

E = 8
D = 4096
T = 128
KB = 512


def _moe_body(x_ref, gw_ref, w_ref, out_ref, comb_ref, xe_ref):
    e = pl.program_id(0)
    k = pl.program_id(1)

    @pl.when((e == 0) & (k == 0))
    def _():
        logits = jnp.dot(x_ref[...], gw_ref[...],
                         preferred_element_type=jnp.float32)
        eio = jax.lax.broadcasted_iota(jnp.int32, (T, E), 1)
        big = jnp.int32(E)
        m1 = jnp.max(logits, axis=1, keepdims=True)
        i1 = jnp.min(jnp.where(logits == m1, eio, big), axis=1, keepdims=True)
        sel1 = eio == i1
        masked = jnp.where(sel1, -jnp.inf, logits)
        m2 = jnp.max(masked, axis=1, keepdims=True)
        i2 = jnp.min(jnp.where(masked == m2, eio, big), axis=1, keepdims=True)
        sel2 = eio == i2
        t = jnp.exp(m2 - m1)
        w1 = 1.0 / (1.0 + t)
        w2 = 1.0 - w1
        comb_ref[...] = jnp.where(sel1, w1, 0.0) + jnp.where(sel2, w2, 0.0)
        out_ref[...] = jnp.zeros_like(out_ref)

    @pl.when(k == 0)
    def _():
        eio = jax.lax.broadcasted_iota(jnp.int32, (1, E), 1)
        c = jnp.sum(comb_ref[...] * (eio == e).astype(jnp.float32),
                    axis=1, keepdims=True)
        xe_ref[...] = (x_ref[...] * c).astype(jnp.bfloat16)

    out_ref[...] += jnp.dot(xe_ref[:, pl.ds(k * KB, KB)], w_ref[0],
                            precision=jax.lax.Precision.DEFAULT,
                            preferred_element_type=jnp.float32)


def kernel(x, gate_w, expert_w):
    return pl.pallas_call(
        _moe_body,
        grid=(E, D // KB),
        in_specs=[
            pl.BlockSpec((T, D), lambda e, k: (0, 0)),
            pl.BlockSpec((D, E), lambda e, k: (0, 0)),
            pl.BlockSpec((1, KB, D), lambda e, k: (e, k, 0)),
        ],
        out_specs=pl.BlockSpec((T, D), lambda e, k: (0, 0)),
        out_shape=jax.ShapeDtypeStruct((T, D), jnp.float32),
        scratch_shapes=[pltpu.VMEM((T, E), jnp.float32),
                        pltpu.VMEM((T, D), jnp.bfloat16)],
    )(x, gate_w, expert_w)
